# baseline (speedup 1.0000x reference)
import jax
import jax.numpy as jnp
from jax.experimental import pallas as pl
from jax.experimental.pallas import tpu as pltpu
from jax.experimental.pallas import tpu_sc as plsc

_W = 128
_PW = 2048
_OFF = 501760


def _tc_prep(ids):
    b, s = ids.shape

    def body(i_ref, h_ref, p_ref):
        x = jnp.swapaxes(i_ref[...], 0, 1)
        hi = (x >= _OFF).astype(jnp.int32)
        h_ref[...] = x - _OFF * hi
        p_ref[...] = hi

    return pl.pallas_call(
        body,
        out_shape=(
            jax.ShapeDtypeStruct((s, b), jnp.int32),
            jax.ShapeDtypeStruct((s, b), jnp.int32),
        ),
    )(ids.astype(jnp.int32))


def _tc_pairs(table_t):
    h, v = table_t.shape
    n_blocks = pl.cdiv(v, _PW)
    half_blocks = _OFF // _PW

    def body(a_ref, b_ref, i_ref, o_ref):
        eye = i_ref[...]
        ta = jax.lax.dot_general(
            a_ref[...].astype(jnp.bfloat16), eye,
            (((0,), (0,)), ((), ())), preferred_element_type=jnp.float32)
        tb = jax.lax.dot_general(
            b_ref[...].astype(jnp.bfloat16), eye,
            (((0,), (0,)), ((), ())), preferred_element_type=jnp.float32)
        o_ref[...] = jnp.concatenate([ta, tb], axis=1)

    return pl.pallas_call(
        body,
        grid=(half_blocks,),
        in_specs=[
            pl.BlockSpec((h, _PW), lambda i: (0, i)),
            pl.BlockSpec(
                (h, _PW),
                lambda i, _hb=half_blocks, _nb=n_blocks: (
                    0, jnp.minimum(i + _hb, _nb - 1))),
            pl.BlockSpec((h, h), lambda i: (0, 0)),
        ],
        out_specs=pl.BlockSpec((_PW, 2 * h), lambda i: (i, 0)),
        out_shape=jax.ShapeDtypeStruct((_OFF, 2 * h), table_t.dtype),
        compiler_params=pltpu.CompilerParams(dimension_semantics=("parallel",)),
    )(table_t, table_t, jnp.eye(h, dtype=jnp.bfloat16))


def _sc_gather(pairs, row_idx):
    s, b = row_idx.shape
    n = s * b
    mesh = plsc.VectorSubcoreMesh(core_axis_name="core", subcore_axis_name="subcore")
    wpr = b // _W

    @pl.kernel(
        out_type=jax.ShapeDtypeStruct((n, 128), pairs.dtype),
        mesh=mesh,
    )
    def gather_kernel(pairs_hbm, i_hbm, o_hbm):
        def body(i_vmem, o_vmem):
            pltpu.sync_copy(pairs_hbm.at[i_vmem.at[0]], o_vmem)

        pltpu.emit_pipeline(
            body,
            grid=(s, wpr),
            in_specs=[pl.BlockSpec((1, _W), lambda i, j: (i, j))],
            out_specs=[pl.BlockSpec((_W, 128), lambda i, j, _wpr=wpr: (i * _wpr + j, 0))],
            core_axis_name=("core", "subcore"),
            dimension_semantics=(pltpu.PARALLEL, pltpu.PARALLEL),
        )(i_hbm, o_hbm)

    return gather_kernel(pairs, row_idx)


def _tc_select_add_t(tmp3, half, pos):
    seq, batch, _ = tmp3.shape
    h = pos.shape[-1]

    def body(t_ref, p_ref, e_ref, i_ref, o_ref):
        x = t_ref[0]
        p2 = jax.lax.broadcast_in_dim(p_ref[0, 0], (batch, h), (0,))
        sel = jnp.where(p2 == 1, x[:, h:], x[:, :h])
        selt = jax.lax.dot_general(
            i_ref[...], sel.astype(jnp.bfloat16),
            (((1,), (1,)), ((), ())), preferred_element_type=jnp.float32)
        o_ref[0] = selt + jax.lax.broadcast_in_dim(e_ref[0, 0], (h, batch), (0,))

    return pl.pallas_call(
        body,
        grid=(seq,),
        in_specs=[
            pl.BlockSpec((1, batch, 2 * h), lambda i: (i, 0, 0)),
            pl.BlockSpec((1, 1, batch), lambda i: (i, 0, 0)),
            pl.BlockSpec((1, 1, h), lambda i: (i, 0, 0)),
            pl.BlockSpec((h, h), lambda i: (0, 0)),
        ],
        out_specs=pl.BlockSpec((1, h, batch), lambda i: (i, 0, 0)),
        out_shape=jax.ShapeDtypeStruct((seq, h, batch), pos.dtype),
        compiler_params=pltpu.CompilerParams(dimension_semantics=("parallel",)),
    )(tmp3, half.reshape(seq, 1, batch), pos.reshape(seq, 1, h),
      jnp.eye(h, dtype=jnp.bfloat16))


def kernel(input_ids, token_table, position_table):
    batch, seq = input_ids.shape
    vocab, hidden = token_table.shape

    row_idx, half = _tc_prep(input_ids)
    table_t = jnp.swapaxes(token_table, 0, 1)
    pairs = _tc_pairs(table_t)

    bounds = [0, 8 * (seq // 16), seq]
    outs = []
    for k in range(len(bounds) - 1):
        sl = slice(bounds[k], bounds[k + 1])
        cs = bounds[k + 1] - bounds[k]
        tmp = _sc_gather(pairs, row_idx[sl])
        tmp3 = tmp.reshape(cs, batch, 2 * hidden)
        outs.append(_tc_select_add_t(tmp3, half[sl], position_table[sl]))
    out_t = jnp.concatenate(outs, axis=0)
    return jnp.swapaxes(out_t, 1, 2)


# speedup vs baseline: 1.1032x; 1.1032x over previous
import jax
import jax.numpy as jnp
from jax.experimental import pallas as pl
from jax.experimental.pallas import tpu as pltpu
from jax.experimental.pallas import tpu_sc as plsc

_W = 128
_PW = 2048
_OFF = 501760


def _tc_prep(ids):
    b, s = ids.shape

    def body(i_ref, h_ref, p_ref):
        x = jnp.swapaxes(i_ref[...], 0, 1)
        hi = (x >= _OFF).astype(jnp.int32)
        h_ref[...] = x - _OFF * hi
        p_ref[...] = hi

    return pl.pallas_call(
        body,
        out_shape=(
            jax.ShapeDtypeStruct((s, b), jnp.int32),
            jax.ShapeDtypeStruct((s, b), jnp.int32),
        ),
    )(ids.astype(jnp.int32))


def _tc_pairs(table_t):
    h, v = table_t.shape
    n_blocks = pl.cdiv(v, _PW)
    half_blocks = _OFF // _PW

    def body(a_ref, b_ref, i_ref, o_ref):
        eye = i_ref[...]
        ta = jax.lax.dot_general(
            a_ref[...].astype(jnp.bfloat16), eye,
            (((0,), (0,)), ((), ())), preferred_element_type=jnp.float32)
        tb = jax.lax.dot_general(
            b_ref[...].astype(jnp.bfloat16), eye,
            (((0,), (0,)), ((), ())), preferred_element_type=jnp.float32)
        o_ref[...] = jnp.concatenate([ta, tb], axis=1)

    return pl.pallas_call(
        body,
        grid=(half_blocks,),
        in_specs=[
            pl.BlockSpec((h, _PW), lambda i: (0, i)),
            pl.BlockSpec(
                (h, _PW),
                lambda i, _hb=half_blocks, _nb=n_blocks: (
                    0, jnp.minimum(i + _hb, _nb - 1))),
            pl.BlockSpec((h, h), lambda i: (0, 0)),
        ],
        out_specs=pl.BlockSpec((_PW, 2 * h), lambda i: (i, 0)),
        out_shape=jax.ShapeDtypeStruct((_OFF, 2 * h), table_t.dtype),
        compiler_params=pltpu.CompilerParams(dimension_semantics=("parallel",)),
    )(table_t, table_t, jnp.eye(h, dtype=jnp.bfloat16))


def _sc_gather(pairs, row_idx):
    s, b = row_idx.shape
    n = s * b
    mesh = plsc.VectorSubcoreMesh(core_axis_name="core", subcore_axis_name="subcore")
    wpr = b // _W

    @pl.kernel(
        out_type=jax.ShapeDtypeStruct((n, 128), pairs.dtype),
        mesh=mesh,
    )
    def gather_kernel(pairs_hbm, i_hbm, o_hbm):
        def body(i_vmem, o_vmem):
            pltpu.sync_copy(pairs_hbm.at[i_vmem.at[0]], o_vmem)

        pltpu.emit_pipeline(
            body,
            grid=(s, wpr),
            in_specs=[pl.BlockSpec((1, _W), lambda i, j: (i, j))],
            out_specs=[pl.BlockSpec((_W, 128), lambda i, j, _wpr=wpr: (i * _wpr + j, 0))],
            core_axis_name=("core", "subcore"),
            dimension_semantics=(pltpu.PARALLEL, pltpu.PARALLEL),
        )(i_hbm, o_hbm)

    return gather_kernel(pairs, row_idx)


def _tc_select_add_t(tmp3, half, pos):
    seq, batch, _ = tmp3.shape
    h = pos.shape[-1]

    sc = max(d for d in (8, 4, 2, 1) if seq % d == 0)

    def body(t_ref, p_ref, e_ref, o_ref):
        x = t_ref[...]
        p3 = jax.lax.broadcast_in_dim(p_ref[...], (sc, batch, h), (0, 1))
        sel = jnp.where(p3 == 1, x[:, :, h:], x[:, :, :h])
        selt = jnp.transpose(sel, (0, 2, 1))
        o_ref[...] = selt + jax.lax.broadcast_in_dim(e_ref[...], (sc, h, batch), (0, 1))

    return pl.pallas_call(
        body,
        grid=(seq // sc,),
        in_specs=[
            pl.BlockSpec((sc, batch, 128), lambda i: (i, 0, 0)),
            pl.BlockSpec((sc, batch), lambda i: (i, 0)),
            pl.BlockSpec((sc, h), lambda i: (i, 0)),
        ],
        out_specs=pl.BlockSpec((sc, h, batch), lambda i: (i, 0, 0)),
        out_shape=jax.ShapeDtypeStruct((seq, h, batch), pos.dtype),
        compiler_params=pltpu.CompilerParams(dimension_semantics=("parallel",)),
    )(tmp3, half, pos)


def kernel(input_ids, token_table, position_table):
    batch, seq = input_ids.shape
    vocab, hidden = token_table.shape

    row_idx, half = _tc_prep(input_ids)
    table_t = jnp.swapaxes(token_table, 0, 1)
    pairs = _tc_pairs(table_t)

    tmp = _sc_gather(pairs, row_idx)
    tmp3 = tmp.reshape(seq, batch, 2 * hidden)
    out_t = _tc_select_add_t(tmp3, half, position_table)
    return jnp.swapaxes(out_t, 1, 2)
